# Initial kernel scaffold; baseline (speedup 1.0000x reference)
import jax
import jax.numpy as jnp
from jax.experimental import pallas as pl


def kernel(latents, embedding_weight):
    raise NotImplementedError("write your pallas kernel here")


# speedup vs baseline: 2.5342x; 2.5342x over previous
import functools

import jax
import jax.numpy as jnp
from jax import lax
from jax.experimental import pallas as pl
from jax.experimental.pallas import tpu as pltpu
from jax.experimental.pallas import tpu_sc as plsc

_K = 1024
_D = 64
_B = 16
_HW = 1024
_N = _B * _HW


def _tc_body(lat_ref, e_ref, inds_ref, loss_ref):
    b = pl.program_id(0)
    lat = lat_ref[0]
    emb = e_ref[...]
    scores = lax.dot_general(
        emb, lat, (((1,), (0,)), ((), ())),
        preferred_element_type=jnp.float32)
    enorm = jnp.sum(emb * emb, axis=1, keepdims=True)
    fnorm = jnp.sum(lat * lat, axis=0, keepdims=True)
    dist = (fnorm + enorm) - 2.0 * scores
    inds_ref[0, 0, :] = jnp.argmin(dist, axis=0).astype(jnp.int32)
    s = jnp.sum(jnp.min(dist, axis=0, keepdims=True), axis=1, keepdims=True)

    @pl.when(b == 0)
    def _init():
        loss_ref[...] = jnp.zeros_like(s)

    loss_ref[...] += s


def _tc_argmin(lat3, emb):
    return pl.pallas_call(
        _tc_body,
        grid=(_B,),
        in_specs=[
            pl.BlockSpec((1, _D, _HW), lambda b: (b, 0, 0)),
            pl.BlockSpec((_K, _D), lambda b: (0, 0)),
        ],
        out_specs=[
            pl.BlockSpec((1, 1, _HW), lambda b: (b, 0, 0)),
            pl.BlockSpec((1, 1), lambda b: (0, 0)),
        ],
        out_shape=[
            jax.ShapeDtypeStruct((_B, 1, _HW), jnp.int32),
            jax.ShapeDtypeStruct((1, 1), jnp.float32),
        ],
    )(lat3, emb)


_NC = 2
_NS = 16
_NW = _NC * _NS
_BPW = _N // _NW

@functools.cache
def _sc_gather_fn():
    mesh = plsc.VectorSubcoreMesh(
        core_axis_name="c", subcore_axis_name="s",
        num_cores=_NC, num_subcores=_NS)

    @functools.partial(
        pl.kernel, mesh=mesh,
        out_type=jax.ShapeDtypeStruct((_N, 128), jnp.float32),
        scratch_types=[
            pltpu.VMEM((_BPW,), jnp.int32),
            pltpu.VMEM((_BPW, 128), jnp.float32),
            pltpu.SemaphoreType.DMA,
        ],
    )
    def _sc_gather(table_hbm, idx_hbm, out_hbm, idx_v, rows_v, sem):
        wid = lax.axis_index("s") * _NC + lax.axis_index("c")
        base = wid * _BPW
        pltpu.sync_copy(idx_hbm.at[pl.ds(base, _BPW)], idx_v)
        pltpu.async_copy(table_hbm.at[idx_v], rows_v, sem).wait()
        pltpu.sync_copy(rows_v, out_hbm.at[pl.ds(base, _BPW)])

    return _sc_gather


def kernel(latents, embedding_weight):
    lat3 = latents.reshape(_B, _D, _HW)
    inds3, losssum = _tc_argmin(lat3, embedding_weight)
    inds = inds3.reshape(_N)
    table = jnp.pad(embedding_weight, ((0, 0), (0, 128 - _D)))
    q = _sc_gather_fn()(table, inds)
    out = q[:, :_D].reshape(_B, 32, 32, _D).transpose(0, 3, 1, 2)
    vq_loss = losssum[0, 0] * (1.25 / _N / _D)
    return out, vq_loss
